# Initial kernel scaffold; baseline (speedup 1.0000x reference)
import jax
import jax.numpy as jnp
from jax.experimental import pallas as pl


def kernel(xyz, xyz_new):
    raise NotImplementedError("write your pallas kernel here")


# speedup vs baseline: 8.9763x; 8.9763x over previous
import functools

import jax
import jax.numpy as jnp
from jax import lax
from jax.experimental import pallas as pl
from jax.experimental.pallas import tpu as pltpu, tpu_sc as plsc

_RADIUS = 0.2
_NSAMPLE = 32
_B = 4
_N = 4096
_NW = 32
_ROWS_PER_W = (_B * _N) // _NW
_W_PER_B = _N // _ROWS_PER_W
_L = 16
_R2 = _RADIUS * _RADIUS


def _sc_ball_query():
    mesh = plsc.VectorSubcoreMesh(core_axis_name="c", subcore_axis_name="s")

    @functools.partial(
        pl.kernel,
        out_type=jax.ShapeDtypeStruct((_B * _N * _NSAMPLE,), jnp.int32),
        mesh=mesh,
        scratch_types=[
            pltpu.VMEM((_N,), jnp.float32),
            pltpu.VMEM((_N,), jnp.float32),
            pltpu.VMEM((_N,), jnp.float32),
            pltpu.VMEM((128,), jnp.int32),
            pltpu.VMEM((_ROWS_PER_W * _NSAMPLE,), jnp.int32),
        ],
        compiler_params=pltpu.CompilerParams(needs_layout_passes=False),
    )
    def k(xyz_hbm, out_hbm, cx, cy, cz, row, outbuf):
        wid = lax.axis_index("c") * 16 + lax.axis_index("s")
        b = wid // _W_PER_B
        r0 = (wid % _W_PER_B) * _ROWS_PER_W

        pltpu.sync_copy(xyz_hbm.at[pl.ds((b * 3 + 0) * _N, _N)], cx)
        pltpu.sync_copy(xyz_hbm.at[pl.ds((b * 3 + 1) * _N, _N)], cy)
        pltpu.sync_copy(xyz_hbm.at[pl.ds((b * 3 + 2) * _N, _N)], cz)

        iota = jnp.arange(_L, dtype=jnp.int32)
        zeros = jnp.zeros((_L,), jnp.int32)

        def per_query(qi, carry):
            q = r0 + qi
            qsplat = jnp.full((_L,), q, jnp.int32)
            qx = plsc.load_gather(cx, [qsplat])
            qy = plsc.load_gather(cy, [qsplat])
            qz = plsc.load_gather(cz, [qsplat])

            def cond(state):
                j, cnt = state
                return jnp.logical_and(j < _N, jnp.max(cnt) < _NSAMPLE)

            def body(state):
                j, cnt = state
                sl = pl.ds(j, _L)
                dx = cx[sl] - qx
                dy = cy[sl] - qy
                dz = cz[sl] - qz
                d2 = (dx * dx + dy * dy) + dz * dz
                m = d2 < _R2
                mi = m.astype(jnp.int32)
                pos = cnt + plsc.cumsum(mi) - 1
                idxv = iota + j
                plsc.store_scatter(row, [pos], idxv, mask=m)
                cnt = cnt + plsc.all_reduce_population_count(m)
                return j + _L, cnt

            _, cnt = lax.while_loop(cond, body, (jnp.int32(0), zeros))

            r0v = row[pl.ds(0, _L)]
            r1v = row[pl.ds(_L, _L)]
            first = jnp.min(jnp.where(iota < cnt, r0v, jnp.int32(1 << 30)))
            o0 = jnp.where(iota < cnt, r0v, first)
            o1 = jnp.where(iota + _L < cnt, r1v, first)
            obase = qi * _NSAMPLE
            outbuf[pl.ds(obase, _L)] = o0
            outbuf[pl.ds(obase + _L, _L)] = o1
            return carry

        lax.fori_loop(0, _ROWS_PER_W, per_query, 0)
        out_off = (b * _N + r0) * _NSAMPLE
        pltpu.sync_copy(outbuf, out_hbm.at[pl.ds(out_off, _ROWS_PER_W * _NSAMPLE)])

    return k


def kernel(xyz, xyz_new):
    del xyz_new
    xyz_flat = jnp.transpose(xyz, (0, 2, 1)).reshape(-1)
    out_flat = _sc_ball_query()(xyz_flat)
    return out_flat.reshape(_B, _N, _NSAMPLE)


# speedup vs baseline: 14.7735x; 1.6458x over previous
import functools

import jax
import jax.numpy as jnp
from jax import lax
from jax.experimental import pallas as pl
from jax.experimental.pallas import tpu as pltpu, tpu_sc as plsc

_RADIUS = 0.2
_NSAMPLE = 32
_B = 4
_N = 4096
_NW = 32
_ROWS_PER_W = (_B * _N) // _NW
_W_PER_B = _N // _ROWS_PER_W
_L = 16
_R2 = _RADIUS * _RADIUS


def _sc_ball_query():
    mesh = plsc.VectorSubcoreMesh(core_axis_name="c", subcore_axis_name="s")

    @functools.partial(
        pl.kernel,
        out_type=jax.ShapeDtypeStruct((_B * _N * _NSAMPLE,), jnp.int32),
        mesh=mesh,
        scratch_types=[
            pltpu.VMEM((_N,), jnp.float32),
            pltpu.VMEM((_N,), jnp.float32),
            pltpu.VMEM((_N,), jnp.float32),
            pltpu.VMEM((128,), jnp.int32),
            pltpu.VMEM((_ROWS_PER_W * _NSAMPLE,), jnp.int32),
        ],
        compiler_params=pltpu.CompilerParams(needs_layout_passes=False),
    )
    def k(xyz_hbm, out_hbm, cx, cy, cz, row, outbuf):
        wid = lax.axis_index("c") * 16 + lax.axis_index("s")
        b = wid // _W_PER_B
        r0 = (wid % _W_PER_B) * _ROWS_PER_W

        pltpu.sync_copy(xyz_hbm.at[pl.ds((b * 3 + 0) * _N, _N)], cx)
        pltpu.sync_copy(xyz_hbm.at[pl.ds((b * 3 + 1) * _N, _N)], cy)
        pltpu.sync_copy(xyz_hbm.at[pl.ds((b * 3 + 2) * _N, _N)], cz)

        iota = jnp.arange(_L, dtype=jnp.int32)
        zeros = jnp.zeros((_L,), jnp.int32)

        def per_query(qi, carry):
            q = r0 + qi
            qsplat = jnp.full((_L,), q, jnp.int32)
            qx = plsc.load_gather(cx, [qsplat])
            qy = plsc.load_gather(cy, [qsplat])
            qz = plsc.load_gather(cz, [qsplat])

            def cond(state):
                j, cnt = state
                cnt_s = lax.squeeze(lax.slice(cnt, (0,), (1,)), (0,))
                return jnp.logical_and(j < _N, cnt_s < _NSAMPLE)

            def body(state):
                j, cnt = state
                for u in range(4):
                    sl = pl.ds(j + u * _L, _L)
                    dx = cx[sl] - qx
                    dy = cy[sl] - qy
                    dz = cz[sl] - qz
                    d2 = (dx * dx + dy * dy) + dz * dz
                    m = d2 < _R2
                    mi = m.astype(jnp.int32)
                    pos = cnt + plsc.cumsum(mi) - 1
                    idxv = iota + (j + u * _L)
                    plsc.store_scatter(row, [pos], idxv, mask=m)
                    cnt = cnt + plsc.all_reduce_population_count(m)
                return j + 4 * _L, cnt

            _, cnt = lax.while_loop(cond, body, (jnp.int32(0), zeros))

            r0v = row[pl.ds(0, _L)]
            r1v = row[pl.ds(_L, _L)]
            first = jnp.min(jnp.where(iota < cnt, r0v, jnp.int32(1 << 30)))
            o0 = jnp.where(iota < cnt, r0v, first)
            o1 = jnp.where(iota + _L < cnt, r1v, first)
            obase = qi * _NSAMPLE
            outbuf[pl.ds(obase, _L)] = o0
            outbuf[pl.ds(obase + _L, _L)] = o1
            return carry

        lax.fori_loop(0, _ROWS_PER_W, per_query, 0)
        out_off = (b * _N + r0) * _NSAMPLE
        pltpu.sync_copy(outbuf, out_hbm.at[pl.ds(out_off, _ROWS_PER_W * _NSAMPLE)])

    return k


def kernel(xyz, xyz_new):
    del xyz_new
    xyz_flat = jnp.transpose(xyz, (0, 2, 1)).reshape(-1)
    out_flat = _sc_ball_query()(xyz_flat)
    return out_flat.reshape(_B, _N, _NSAMPLE)


# speedup vs baseline: 18.3609x; 1.2428x over previous
import functools

import jax
import jax.numpy as jnp
from jax import lax
from jax.experimental import pallas as pl
from jax.experimental.pallas import tpu as pltpu, tpu_sc as plsc

_RADIUS = 0.2
_NSAMPLE = 32
_B = 4
_N = 4096
_NW = 32
_ROWS_PER_W = (_B * _N) // _NW
_W_PER_B = _N // _ROWS_PER_W
_L = 16
_R2 = _RADIUS * _RADIUS


def _sc_ball_query():
    mesh = plsc.VectorSubcoreMesh(core_axis_name="c", subcore_axis_name="s")

    @functools.partial(
        pl.kernel,
        out_type=jax.ShapeDtypeStruct((_B * _N * _NSAMPLE,), jnp.int32),
        mesh=mesh,
        scratch_types=[
            pltpu.VMEM((_N,), jnp.float32),
            pltpu.VMEM((_N,), jnp.float32),
            pltpu.VMEM((_N,), jnp.float32),
            pltpu.VMEM((128,), jnp.int32),
            pltpu.VMEM((_ROWS_PER_W * _NSAMPLE,), jnp.int32),
        ],
        compiler_params=pltpu.CompilerParams(needs_layout_passes=False),
    )
    def k(xyz_hbm, out_hbm, cx, cy, cz, row, outbuf):
        wid = lax.axis_index("c") * 16 + lax.axis_index("s")
        b = wid // _W_PER_B
        r0 = (wid % _W_PER_B) * _ROWS_PER_W

        pltpu.sync_copy(xyz_hbm.at[pl.ds((b * 3 + 0) * _N, _N)], cx)
        pltpu.sync_copy(xyz_hbm.at[pl.ds((b * 3 + 1) * _N, _N)], cy)
        pltpu.sync_copy(xyz_hbm.at[pl.ds((b * 3 + 2) * _N, _N)], cz)

        iota = jnp.arange(_L, dtype=jnp.int32)
        zeros = jnp.zeros((_L,), jnp.int32)

        def per_query(qi, carry):
            q = r0 + qi
            qsplat = jnp.full((_L,), q, jnp.int32)
            qx = plsc.load_gather(cx, [qsplat])
            qy = plsc.load_gather(cy, [qsplat])
            qz = plsc.load_gather(cz, [qsplat])

            def cond(state):
                j, cnt = state
                return jnp.logical_and(j < _N, cnt < _NSAMPLE)

            def body(state):
                j, cnt = state
                for u in range(4):
                    sl = pl.ds(j + u * _L, _L)
                    dx = cx[sl] - qx
                    dy = cy[sl] - qy
                    dz = cz[sl] - qz
                    d2 = (dx * dx + dy * dy) + dz * dz
                    m = d2 < _R2
                    idxv = iota + (j + u * _L)
                    plsc.store_compressed(row.at[pl.ds(cnt, _L)], idxv, mask=m)
                    pc = plsc.all_reduce_population_count(m)
                    cnt = cnt + lax.squeeze(lax.slice(pc, (0,), (1,)), (0,))
                return j + 4 * _L, cnt

            _, cnt = lax.while_loop(cond, body, (jnp.int32(0), jnp.int32(0)))

            r0v = row[pl.ds(0, _L)]
            r1v = row[pl.ds(_L, _L)]
            first = jnp.min(jnp.where(iota < cnt, r0v, jnp.int32(1 << 30)))
            o0 = jnp.where(iota < cnt, r0v, first)
            o1 = jnp.where(iota + _L < cnt, r1v, first)
            obase = qi * _NSAMPLE
            outbuf[pl.ds(obase, _L)] = o0
            outbuf[pl.ds(obase + _L, _L)] = o1
            return carry

        lax.fori_loop(0, _ROWS_PER_W, per_query, 0)
        out_off = (b * _N + r0) * _NSAMPLE
        pltpu.sync_copy(outbuf, out_hbm.at[pl.ds(out_off, _ROWS_PER_W * _NSAMPLE)])

    return k


def kernel(xyz, xyz_new):
    del xyz_new
    xyz_flat = jnp.transpose(xyz, (0, 2, 1)).reshape(-1)
    out_flat = _sc_ball_query()(xyz_flat)
    return out_flat.reshape(_B, _N, _NSAMPLE)


# speedup vs baseline: 36.2092x; 1.9721x over previous
import functools

import jax
import jax.numpy as jnp
from jax import lax
from jax.experimental import pallas as pl
from jax.experimental.pallas import tpu as pltpu, tpu_sc as plsc

_RADIUS = 0.2
_NSAMPLE = 32
_B = 4
_N = 4096
_NW = 32
_ROWS_PER_W = (_B * _N) // _NW
_W_PER_B = _N // _ROWS_PER_W
_L = 16
_R2 = _RADIUS * _RADIUS


def _sc_ball_query():
    mesh = plsc.VectorSubcoreMesh(core_axis_name="c", subcore_axis_name="s")

    @functools.partial(
        pl.kernel,
        out_type=jax.ShapeDtypeStruct((_B * _N * _NSAMPLE,), jnp.int32),
        mesh=mesh,
        scratch_types=[
            pltpu.VMEM((_N,), jnp.float32),
            pltpu.VMEM((_N,), jnp.float32),
            pltpu.VMEM((_N,), jnp.float32),
            pltpu.VMEM((128,), jnp.int32),
            pltpu.VMEM((_ROWS_PER_W * _NSAMPLE,), jnp.int32),
        ],
        compiler_params=pltpu.CompilerParams(needs_layout_passes=False),
    )
    def k(xyz_hbm, out_hbm, cx, cy, cz, row, outbuf):
        wid = lax.axis_index("c") * 16 + lax.axis_index("s")
        b = wid // _W_PER_B
        r0 = (wid % _W_PER_B) * _ROWS_PER_W

        pltpu.sync_copy(xyz_hbm.at[pl.ds((b * 3 + 0) * _N, _N)], cx)
        pltpu.sync_copy(xyz_hbm.at[pl.ds((b * 3 + 1) * _N, _N)], cy)
        pltpu.sync_copy(xyz_hbm.at[pl.ds((b * 3 + 2) * _N, _N)], cz)

        iota = jnp.arange(_L, dtype=jnp.int32)
        zeros = jnp.zeros((_L,), jnp.int32)

        def per_query(qi, carry):
            q = r0 + qi
            qsplat = jnp.full((_L,), q, jnp.int32)
            qx = plsc.load_gather(cx, [qsplat])
            qy = plsc.load_gather(cy, [qsplat])
            qz = plsc.load_gather(cz, [qsplat])

            def cond(state):
                j, cnt = state
                return jnp.logical_and(j < _N, cnt < _NSAMPLE)

            def body(state):
                j, cnt = state
                ms, idxs, pcs = [], [], []
                for u in range(4):
                    sl = pl.ds(j + u * _L, _L)
                    dx = cx[sl] - qx
                    dy = cy[sl] - qy
                    dz = cz[sl] - qz
                    d2 = (dx * dx + dy * dy) + dz * dz
                    m = d2 < _R2
                    ms.append(m)
                    idxs.append(iota + (j + u * _L))
                    pc = plsc.all_reduce_population_count(m)
                    pcs.append(lax.squeeze(lax.slice(pc, (0,), (1,)), (0,)))
                for u in range(4):
                    plsc.store_compressed(row.at[pl.ds(cnt, _L)], idxs[u],
                                          mask=ms[u])
                    cnt = cnt + pcs[u]
                return j + 4 * _L, cnt

            _, cnt = lax.while_loop(cond, body, (jnp.int32(0), jnp.int32(0)))

            r0v = row[pl.ds(0, _L)]
            r1v = row[pl.ds(_L, _L)]
            first = jnp.min(jnp.where(iota < cnt, r0v, jnp.int32(1 << 30)))
            o0 = jnp.where(iota < cnt, r0v, first)
            o1 = jnp.where(iota + _L < cnt, r1v, first)
            obase = qi * _NSAMPLE
            outbuf[pl.ds(obase, _L)] = o0
            outbuf[pl.ds(obase + _L, _L)] = o1
            return carry

        lax.fori_loop(0, _ROWS_PER_W, per_query, 0)
        out_off = (b * _N + r0) * _NSAMPLE
        pltpu.sync_copy(outbuf, out_hbm.at[pl.ds(out_off, _ROWS_PER_W * _NSAMPLE)])

    return k


def kernel(xyz, xyz_new):
    del xyz_new
    xyz_flat = jnp.transpose(xyz, (0, 2, 1)).reshape(-1)
    out_flat = _sc_ball_query()(xyz_flat)
    return out_flat.reshape(_B, _N, _NSAMPLE)


# speedup vs baseline: 50.0646x; 1.3826x over previous
import functools

import jax
import jax.numpy as jnp
from jax import lax
from jax.experimental import pallas as pl
from jax.experimental.pallas import tpu as pltpu, tpu_sc as plsc

_RADIUS = 0.2
_NSAMPLE = 32
_B = 4
_N = 4096
_NW = 32
_ROWS_PER_W = (_B * _N) // _NW
_W_PER_B = _N // _ROWS_PER_W
_L = 16
_R2 = _RADIUS * _RADIUS


def _sc_ball_query():
    mesh = plsc.VectorSubcoreMesh(core_axis_name="c", subcore_axis_name="s")

    @functools.partial(
        pl.kernel,
        out_type=jax.ShapeDtypeStruct((_B * _N * _NSAMPLE,), jnp.int32),
        mesh=mesh,
        scratch_types=[
            pltpu.VMEM((_N,), jnp.float32),
            pltpu.VMEM((_N,), jnp.float32),
            pltpu.VMEM((_N,), jnp.float32),
            pltpu.VMEM((192,), jnp.int32),
            pltpu.VMEM((_ROWS_PER_W * _NSAMPLE,), jnp.int32),
        ],
        compiler_params=pltpu.CompilerParams(needs_layout_passes=False),
    )
    def k(xyz_hbm, out_hbm, cx, cy, cz, row, outbuf):
        wid = lax.axis_index("c") * 16 + lax.axis_index("s")
        b = wid // _W_PER_B
        r0 = (wid % _W_PER_B) * _ROWS_PER_W

        pltpu.sync_copy(xyz_hbm.at[pl.ds((b * 3 + 0) * _N, _N)], cx)
        pltpu.sync_copy(xyz_hbm.at[pl.ds((b * 3 + 1) * _N, _N)], cy)
        pltpu.sync_copy(xyz_hbm.at[pl.ds((b * 3 + 2) * _N, _N)], cz)

        iota = jnp.arange(_L, dtype=jnp.int32)
        zeros = jnp.zeros((_L,), jnp.int32)

        def per_query(qi, carry):
            q = r0 + qi
            qsplat = jnp.full((_L,), q, jnp.int32)
            qx = plsc.load_gather(cx, [qsplat])
            qy = plsc.load_gather(cy, [qsplat])
            qz = plsc.load_gather(cz, [qsplat])

            def cond(state):
                j, cnt = state
                return jnp.logical_and(j < _N, cnt < _NSAMPLE)

            def body(state):
                j, cnt = state
                ms, idxs, pcs = [], [], []
                for u in range(8):
                    sl = pl.ds(j + u * _L, _L)
                    dx = cx[sl] - qx
                    dy = cy[sl] - qy
                    dz = cz[sl] - qz
                    d2 = (dx * dx + dy * dy) + dz * dz
                    m = d2 < _R2
                    ms.append(m)
                    idxs.append(iota + (j + u * _L))
                    pc = plsc.all_reduce_population_count(m)
                    pcs.append(lax.squeeze(lax.slice(pc, (0,), (1,)), (0,)))
                for u in range(8):
                    plsc.store_compressed(row.at[pl.ds(cnt, _L)], idxs[u],
                                          mask=ms[u])
                    cnt = cnt + pcs[u]
                return j + 8 * _L, cnt

            _, cnt = lax.while_loop(cond, body, (jnp.int32(0), jnp.int32(0)))

            r0v = row[pl.ds(0, _L)]
            r1v = row[pl.ds(_L, _L)]
            first = jnp.min(jnp.where(iota < cnt, r0v, jnp.int32(1 << 30)))
            o0 = jnp.where(iota < cnt, r0v, first)
            o1 = jnp.where(iota + _L < cnt, r1v, first)
            obase = qi * _NSAMPLE
            outbuf[pl.ds(obase, _L)] = o0
            outbuf[pl.ds(obase + _L, _L)] = o1
            return carry

        lax.fori_loop(0, _ROWS_PER_W, per_query, 0)
        out_off = (b * _N + r0) * _NSAMPLE
        pltpu.sync_copy(outbuf, out_hbm.at[pl.ds(out_off, _ROWS_PER_W * _NSAMPLE)])

    return k


def kernel(xyz, xyz_new):
    del xyz_new
    xyz_flat = jnp.transpose(xyz, (0, 2, 1)).reshape(-1)
    out_flat = _sc_ball_query()(xyz_flat)
    return out_flat.reshape(_B, _N, _NSAMPLE)


# speedup vs baseline: 60.9419x; 1.2173x over previous
import functools

import jax
import jax.numpy as jnp
from jax import lax
from jax.experimental import pallas as pl
from jax.experimental.pallas import tpu as pltpu, tpu_sc as plsc

_RADIUS = 0.2
_NSAMPLE = 32
_B = 4
_N = 4096
_NW = 32
_ROWS_PER_W = (_B * _N) // _NW
_W_PER_B = _N // _ROWS_PER_W
_L = 16
_R2 = _RADIUS * _RADIUS


def _sc_ball_query():
    mesh = plsc.VectorSubcoreMesh(core_axis_name="c", subcore_axis_name="s")

    @functools.partial(
        pl.kernel,
        out_type=jax.ShapeDtypeStruct((_B * _N * _NSAMPLE,), jnp.int32),
        mesh=mesh,
        scratch_types=[
            pltpu.VMEM((_N,), jnp.float32),
            pltpu.VMEM((_N,), jnp.float32),
            pltpu.VMEM((_N,), jnp.float32),
            pltpu.VMEM((320,), jnp.int32),
            pltpu.VMEM((_ROWS_PER_W * _NSAMPLE,), jnp.int32),
        ],
        compiler_params=pltpu.CompilerParams(needs_layout_passes=False),
    )
    def k(xyz_hbm, out_hbm, cx, cy, cz, row, outbuf):
        wid = lax.axis_index("c") * 16 + lax.axis_index("s")
        b = wid // _W_PER_B
        r0 = (wid % _W_PER_B) * _ROWS_PER_W

        pltpu.sync_copy(xyz_hbm.at[pl.ds((b * 3 + 0) * _N, _N)], cx)
        pltpu.sync_copy(xyz_hbm.at[pl.ds((b * 3 + 1) * _N, _N)], cy)
        pltpu.sync_copy(xyz_hbm.at[pl.ds((b * 3 + 2) * _N, _N)], cz)

        iota = jnp.arange(_L, dtype=jnp.int32)
        zeros = jnp.zeros((_L,), jnp.int32)

        def per_query(qi, carry):
            q = r0 + qi
            qsplat = jnp.full((_L,), q, jnp.int32)
            qx = plsc.load_gather(cx, [qsplat])
            qy = plsc.load_gather(cy, [qsplat])
            qz = plsc.load_gather(cz, [qsplat])

            def cond(state):
                j, cnt = state
                return jnp.logical_and(j < _N, cnt < _NSAMPLE)

            def body(state):
                j, cnt = state
                ms, idxs, pcs = [], [], []
                for u in range(16):
                    sl = pl.ds(j + u * _L, _L)
                    dx = cx[sl] - qx
                    dy = cy[sl] - qy
                    dz = cz[sl] - qz
                    d2 = (dx * dx + dy * dy) + dz * dz
                    m = d2 < _R2
                    ms.append(m)
                    idxs.append(iota + (j + u * _L))
                    pc = plsc.all_reduce_population_count(m)
                    pcs.append(lax.squeeze(lax.slice(pc, (0,), (1,)), (0,)))
                for u in range(16):
                    plsc.store_compressed(row.at[pl.ds(cnt, _L)], idxs[u],
                                          mask=ms[u])
                    cnt = cnt + pcs[u]
                return j + 16 * _L, cnt

            _, cnt = lax.while_loop(cond, body, (jnp.int32(0), jnp.int32(0)))

            r0v = row[pl.ds(0, _L)]
            r1v = row[pl.ds(_L, _L)]
            first = jnp.min(jnp.where(iota < cnt, r0v, jnp.int32(1 << 30)))
            o0 = jnp.where(iota < cnt, r0v, first)
            o1 = jnp.where(iota + _L < cnt, r1v, first)
            obase = qi * _NSAMPLE
            outbuf[pl.ds(obase, _L)] = o0
            outbuf[pl.ds(obase + _L, _L)] = o1
            return carry

        lax.fori_loop(0, _ROWS_PER_W, per_query, 0)
        out_off = (b * _N + r0) * _NSAMPLE
        pltpu.sync_copy(outbuf, out_hbm.at[pl.ds(out_off, _ROWS_PER_W * _NSAMPLE)])

    return k


def kernel(xyz, xyz_new):
    del xyz_new
    xyz_flat = jnp.transpose(xyz, (0, 2, 1)).reshape(-1)
    out_flat = _sc_ball_query()(xyz_flat)
    return out_flat.reshape(_B, _N, _NSAMPLE)
